# Initial kernel scaffold; baseline (speedup 1.0000x reference)
import jax
import jax.numpy as jnp
from jax.experimental import pallas as pl


def kernel(input, emb2, emb3):
    raise NotImplementedError("write your pallas kernel here")


# speedup vs baseline: 113.0769x; 113.0769x over previous
import functools

import jax
import jax.numpy as jnp
from jax.experimental import pallas as pl
from jax.experimental.pallas import tpu as pltpu

_B, _F, _D = 16384, 26, 64
_BLK = 512


def _body(x_ref, e2_ref, e3_ref, o_ref):
    x = x_ref[...]
    v = x - ((x >> 1) & 0x5555)
    v = (v & 0x3333) + ((v >> 2) & 0x3333)
    v = (v + (v >> 4)) & 0x0F0F
    p = (v + (v >> 8)) & 0x1F
    c1 = jnp.zeros_like(x)
    c2 = jnp.zeros_like(x)
    y = x
    for _ in range(11):
        d = y % 3
        c1 += (d == 1).astype(jnp.int32)
        c2 += (d == 2).astype(jnp.int32)
        y = y // 3
    e2 = e2_ref[...]
    e3 = e3_ref[...]
    base2 = 16.0 * e2[0]
    diff2 = e2[1] - e2[0]
    base3 = 16.0 * e3[0]
    d31 = e3[1] - e3[0]
    d32 = e3[2] - e3[0]
    pf = p.astype(jnp.float32)[..., None]
    c1f = c1.astype(jnp.float32)[..., None]
    c2f = c2.astype(jnp.float32)[..., None]
    out2 = base2 + pf * diff2
    out3 = base3 + c1f * d31 + c2f * d32
    o_ref[...] = jnp.concatenate([out2, out3], axis=-1)


@jax.jit
def kernel(input, emb2, emb3):
    grid = (_B // _BLK,)
    return pl.pallas_call(
        _body,
        grid=grid,
        in_specs=[
            pl.BlockSpec((_BLK, _F), lambda i: (i, 0)),
            pl.BlockSpec((2, _D), lambda i: (0, 0)),
            pl.BlockSpec((3, _D), lambda i: (0, 0)),
        ],
        out_specs=pl.BlockSpec((_BLK, _F, 2 * _D), lambda i: (i, 0, 0)),
        out_shape=jax.ShapeDtypeStruct((_B, _F, 2 * _D), jnp.float32),
    )(input, emb2, emb3)
